# Initial kernel scaffold; baseline (speedup 1.0000x reference)
import jax
import jax.numpy as jnp
from jax.experimental import pallas as pl


def kernel(bbox, position_ids, tab_x, tab_y, tab_h, tab_w, seq_tab, W, b, gamma, beta):
    raise NotImplementedError("write your pallas kernel here")


# speedup vs baseline: 1.8619x; 1.8619x over previous
import functools

import jax
import jax.numpy as jnp
from jax import lax
from jax.experimental import pallas as pl
from jax.experimental.pallas import tpu as pltpu
from jax.experimental.pallas import tpu_sc as plsc

HID = 768
DPC = 128
LD = 192
ROWS_PER_TAB = 1024
N_TAB = 6 * ROWS_PER_TAB
NTOK = 4 * 2048
EPS = 1e-12

NC, NS, LANES = 2, 16, 16
NW = NC * NS
TOK_PW = NTOK // NW
T = 64
NCHUNK = TOK_PW // T
ND = LD // LANES


def _proj_body(tabx, taby, tabh, tabw, w_ref, b_ref, out):
    def dot(a, lo):
        return lax.dot_general(
            a[...], w_ref[pl.ds(lo, DPC), :],
            (((1,), (0,)), ((), ())),
            preferred_element_type=jnp.float32,
            precision=lax.Precision.HIGHEST,
        )

    out[pl.ds(0 * ROWS_PER_TAB, ROWS_PER_TAB), :] = dot(tabx, 0 * DPC)
    out[pl.ds(1 * ROWS_PER_TAB, ROWS_PER_TAB), :] = dot(taby, 1 * DPC)
    out[pl.ds(2 * ROWS_PER_TAB, ROWS_PER_TAB), :] = dot(tabx, 2 * DPC)
    out[pl.ds(3 * ROWS_PER_TAB, ROWS_PER_TAB), :] = dot(taby, 3 * DPC)
    out[pl.ds(4 * ROWS_PER_TAB, ROWS_PER_TAB), :] = dot(tabh, 4 * DPC)
    out[pl.ds(5 * ROWS_PER_TAB, ROWS_PER_TAB), :] = (
        dot(tabw, 5 * DPC) + b_ref[...][None, :]
    )


def _proj(tabx, taby, tabh, tabw, W, b):
    return pl.pallas_call(
        _proj_body,
        out_shape=jax.ShapeDtypeStruct((N_TAB, LD), jnp.float32),
    )(tabx, taby, tabh, tabw, W, b)


def _lane_sum(x):
    idx = lax.iota(jnp.int32, LANES)
    dnums = lax.GatherDimensionNumbers(
        offset_dims=(), collapsed_slice_dims=(0,), start_index_map=(0,))
    for sh in (1, 2, 4, 8):
        perm = lax.gather(
            x, (idx ^ sh)[:, None], dnums, (1,),
            mode=lax.GatherScatterMode.PROMISE_IN_BOUNDS)
        x = x + perm
    return x


_mesh = plsc.VectorSubcoreMesh(core_axis_name="c", subcore_axis_name="s")


@functools.partial(
    pl.kernel,
    out_type=jax.ShapeDtypeStruct((NTOK, LD), jnp.float32),
    mesh=_mesh,
    scratch_types=[
        pltpu.VMEM((5, T), jnp.int32),
        pltpu.VMEM((7, T), jnp.int32),
        pltpu.VMEM((7, T, LD), jnp.float32),
        pltpu.VMEM((T, LD), jnp.float32),
        pltpu.VMEM((2, LD), jnp.float32),
        pltpu.SemaphoreType.DMA,
    ],
    compiler_params=pltpu.CompilerParams(use_tc_tiling_on_sc=False),
)
def _sc_body(ptab, seq, bx0, bx1, bx2, bx3, pos, gamma, beta, out,
             braw, idxs, rows, outv, gb, sem):
    wid = lax.axis_index("s") * NC + lax.axis_index("c")

    pltpu.sync_copy(gamma, gb.at[0])
    pltpu.sync_copy(beta, gb.at[1])

    for c in range(NCHUNK):
        base = wid * TOK_PW + c * T

        for k, src in enumerate((bx0, bx1, bx2, bx3, pos)):
            pltpu.sync_copy(src.at[pl.ds(base, T)], braw.at[k])

        for i in range(T // LANES):
            sl = pl.ds(i * LANES, LANES)
            b0 = braw[0, sl]
            b1 = braw[1, sl]
            b2 = braw[2, sl]
            b3 = braw[3, sl]
            idxs[0, sl] = b0
            idxs[1, sl] = b1 + ROWS_PER_TAB
            idxs[2, sl] = b2 + 2 * ROWS_PER_TAB
            idxs[3, sl] = b3 + 3 * ROWS_PER_TAB
            idxs[4, sl] = (b3 - b1) + 4 * ROWS_PER_TAB
            idxs[5, sl] = (b2 - b0) + 5 * ROWS_PER_TAB
            idxs[6, sl] = braw[4, sl]

        cps = [pltpu.async_copy(ptab.at[idxs.at[j]], rows.at[j], sem)
               for j in range(6)]
        cps.append(pltpu.async_copy(seq.at[idxs.at[6]], rows.at[6], sem))
        for cp in cps:
            cp.wait()

        def token_body(t, carry):
            xs = []
            s_acc = None
            q_acc = None
            for d in range(ND):
                sl = pl.ds(d * LANES, LANES)
                x = rows[0, t, sl]
                for j in range(1, 7):
                    x = x + rows[j, t, sl]
                xs.append(x)
                s_acc = x if d == 0 else s_acc + x
                q_acc = x * x if d == 0 else q_acc + x * x
            inv_n = jnp.float32(1.0 / LD)
            s = _lane_sum(s_acc)
            q = _lane_sum(q_acc)
            mu = s * inv_n
            var = q * inv_n - mu * mu
            x0 = var + jnp.float32(EPS)
            ii = lax.bitcast_convert_type(x0, jnp.int32)
            ii = jnp.int32(0x5F3759DF) - lax.shift_right_logical(ii, 1)
            y = lax.bitcast_convert_type(ii, jnp.float32)
            for _ in range(3):
                y = y * (jnp.float32(1.5) - jnp.float32(0.5) * x0 * y * y)
            for d in range(ND):
                sl = pl.ds(d * LANES, LANES)
                outv[t, sl] = (xs[d] - mu) * y * gb[0, sl] + gb[1, sl]
            return carry

        lax.fori_loop(0, T, token_body, 0)
        pltpu.sync_copy(outv, out.at[pl.ds(base, T)])


@jax.jit
def kernel(bbox, position_ids, tab_x, tab_y, tab_h, tab_w, seq_tab, W, b,
           gamma, beta):
    Bb, Ss = position_ids.shape
    n = Bb * Ss
    bb = bbox.astype(jnp.int32).reshape(n, 4)
    pos = position_ids.astype(jnp.int32).reshape(n)
    ptab = _proj(tab_x, tab_y, tab_h, tab_w, W, b)
    out = _sc_body(ptab, seq_tab,
                   bb[:, 0], bb[:, 1], bb[:, 2], bb[:, 3], pos,
                   gamma, beta)
    return out.reshape(Bb, Ss, LD)


# speedup vs baseline: 2.5081x; 1.3471x over previous
import functools

import jax
import jax.numpy as jnp
from jax import lax
from jax.experimental import pallas as pl
from jax.experimental.pallas import tpu as pltpu
from jax.experimental.pallas import tpu_sc as plsc

HID = 768
DPC = 128
LD = 192
ROWS_PER_TAB = 1024
N_TAB = 6 * ROWS_PER_TAB
NTOK = 4 * 2048
EPS = 1e-12

NC, NS, LANES = 2, 16, 16
NW = NC * NS
TOK_PW = NTOK // NW
T = 32
NCHUNK = TOK_PW // T
ND = LD // LANES


def _proj_body(tabx, taby, tabh, tabw, w_ref, b_ref, out):
    def dot(a, lo):
        return lax.dot_general(
            a[...], w_ref[pl.ds(lo, DPC), :],
            (((1,), (0,)), ((), ())),
            preferred_element_type=jnp.float32,
            precision=lax.Precision.HIGHEST,
        )

    out[pl.ds(0 * ROWS_PER_TAB, ROWS_PER_TAB), :] = dot(tabx, 0 * DPC)
    out[pl.ds(1 * ROWS_PER_TAB, ROWS_PER_TAB), :] = dot(taby, 1 * DPC)
    out[pl.ds(2 * ROWS_PER_TAB, ROWS_PER_TAB), :] = dot(tabx, 2 * DPC)
    out[pl.ds(3 * ROWS_PER_TAB, ROWS_PER_TAB), :] = dot(taby, 3 * DPC)
    out[pl.ds(4 * ROWS_PER_TAB, ROWS_PER_TAB), :] = dot(tabh, 4 * DPC)
    out[pl.ds(5 * ROWS_PER_TAB, ROWS_PER_TAB), :] = (
        dot(tabw, 5 * DPC) + b_ref[...][None, :]
    )


def _proj(tabx, taby, tabh, tabw, W, b):
    return pl.pallas_call(
        _proj_body,
        out_shape=jax.ShapeDtypeStruct((N_TAB, LD), jnp.float32),
    )(tabx, taby, tabh, tabw, W, b)


def _lane_sum(x):
    idx = lax.iota(jnp.int32, LANES)
    dnums = lax.GatherDimensionNumbers(
        offset_dims=(), collapsed_slice_dims=(0,), start_index_map=(0,))
    for sh in (1, 2, 4, 8):
        perm = lax.gather(
            x, (idx ^ sh)[:, None], dnums, (1,),
            mode=lax.GatherScatterMode.PROMISE_IN_BOUNDS)
        x = x + perm
    return x


_mesh = plsc.VectorSubcoreMesh(core_axis_name="c", subcore_axis_name="s")


@functools.partial(
    pl.kernel,
    out_type=jax.ShapeDtypeStruct((NTOK, LD), jnp.float32),
    mesh=_mesh,
    scratch_types=[
        pltpu.VMEM((TOK_PW * 4,), jnp.int32),
        pltpu.VMEM((TOK_PW,), jnp.int32),
        pltpu.VMEM((7, TOK_PW), jnp.int32),
        pltpu.VMEM((2, 7, T, LD), jnp.float32),
        pltpu.VMEM((2, T, LD), jnp.float32),
        pltpu.VMEM((2, LD), jnp.float32),
        pltpu.SemaphoreType.DMA,
        pltpu.SemaphoreType.DMA,
        pltpu.SemaphoreType.DMA,
        pltpu.SemaphoreType.DMA,
    ],
    compiler_params=pltpu.CompilerParams(
        use_tc_tiling_on_sc=False, needs_layout_passes=False),
)
def _sc_body(ptab, seq, bbf, pos, gamma, beta, out,
             braw, posb, idxs, rows, outv, gb, sem_a, sem_b, sem_oa, sem_ob):
    wid = lax.axis_index("s") * NC + lax.axis_index("c")
    wbase = wid * TOK_PW

    pltpu.sync_copy(gamma, gb.at[0])
    pltpu.sync_copy(beta, gb.at[1])
    pltpu.sync_copy(bbf.at[pl.ds(wbase * 4, TOK_PW * 4)], braw)
    pltpu.sync_copy(pos.at[pl.ds(wbase, TOK_PW)], posb)

    lane4 = lax.iota(jnp.int32, LANES) * 4
    for i in range(TOK_PW // LANES):
        sl = pl.ds(i * LANES, LANES)
        b0 = plsc.load_gather(braw, [lane4 + (i * 4 * LANES + 0)])
        b1 = plsc.load_gather(braw, [lane4 + (i * 4 * LANES + 1)])
        b2 = plsc.load_gather(braw, [lane4 + (i * 4 * LANES + 2)])
        b3 = plsc.load_gather(braw, [lane4 + (i * 4 * LANES + 3)])
        idxs[0, sl] = b0
        idxs[1, sl] = b1 + ROWS_PER_TAB
        idxs[2, sl] = b2 + 2 * ROWS_PER_TAB
        idxs[3, sl] = b3 + 3 * ROWS_PER_TAB
        idxs[4, sl] = (b3 - b1) + 4 * ROWS_PER_TAB
        idxs[5, sl] = (b2 - b0) + 5 * ROWS_PER_TAB
        idxs[6, sl] = posb[sl]

    row_sems = (sem_a, sem_b)
    out_sems = (sem_oa, sem_ob)

    def fire(c):
        buf = c % 2
        cps = []
        for j in range(7):
            src = ptab if j < 6 else seq
            cps.append(pltpu.async_copy(
                src.at[idxs.at[j, pl.ds(c * T, T)]],
                rows.at[buf, j], row_sems[buf]))
        return cps

    g_regs = [gb[0, pl.ds(d * LANES, LANES)] for d in range(ND)]
    bt_regs = [gb[1, pl.ds(d * LANES, LANES)] for d in range(ND)]

    pending = {0: fire(0)}
    out_cps = {}
    for c in range(NCHUNK):
        buf = c % 2
        if c + 1 < NCHUNK:
            pending[c + 1] = fire(c + 1)
        for cp in pending.pop(c):
            cp.wait()
        if c >= 2:
            out_cps.pop(c - 2).wait()

        def token_body(t, carry, buf=buf):
            xs = []
            s_acc = None
            q_acc = None
            for d in range(ND):
                sl = pl.ds(d * LANES, LANES)
                x = rows[buf, 0, t, sl]
                for j in range(1, 7):
                    x = x + rows[buf, j, t, sl]
                xs.append(x)
                s_acc = x if d == 0 else s_acc + x
                q_acc = x * x if d == 0 else q_acc + x * x
            inv_n = jnp.float32(1.0 / LD)
            s = _lane_sum(s_acc)
            q = _lane_sum(q_acc)
            mu = s * inv_n
            var = q * inv_n - mu * mu
            x0 = var + jnp.float32(EPS)
            ii = lax.bitcast_convert_type(x0, jnp.int32)
            ii = jnp.int32(0x5F3759DF) - lax.shift_right_logical(ii, 1)
            y = lax.bitcast_convert_type(ii, jnp.float32)
            for _ in range(3):
                y = y * (jnp.float32(1.5) - jnp.float32(0.5) * x0 * y * y)
            for d in range(ND):
                sl = pl.ds(d * LANES, LANES)
                outv[buf, t, sl] = (xs[d] - mu) * y * g_regs[d] + bt_regs[d]
            return carry

        lax.fori_loop(0, T, token_body, 0)
        out_cps[c] = pltpu.async_copy(
            outv.at[buf], out.at[pl.ds(wbase + c * T, T)], out_sems[buf])

    out_cps.pop(NCHUNK - 2).wait()
    out_cps.pop(NCHUNK - 1).wait()


@jax.jit
def kernel(bbox, position_ids, tab_x, tab_y, tab_h, tab_w, seq_tab, W, b,
           gamma, beta):
    Bb, Ss = position_ids.shape
    n = Bb * Ss
    bbf = bbox.astype(jnp.int32).reshape(n * 4)
    pos = position_ids.astype(jnp.int32).reshape(n)
    ptab = _proj(tab_x, tab_y, tab_h, tab_w, W, b)
    out = _sc_body(ptab, seq_tab, bbf, pos, gamma, beta)
    return out.reshape(Bb, Ss, LD)


# speedup vs baseline: 2.7416x; 1.0931x over previous
import functools

import jax
import jax.numpy as jnp
from jax import lax
from jax.experimental import pallas as pl
from jax.experimental.pallas import tpu as pltpu
from jax.experimental.pallas import tpu_sc as plsc

HID = 768
DPC = 128
LD = 192
LDP = 256
ROWS_PER_TAB = 1024
SEQ_BASE = 6 * ROWS_PER_TAB
N_TAB = SEQ_BASE + 2048
B_SZ, S_SZ = 4, 2048
NTOK = B_SZ * S_SZ
EPS = 1e-12

NC, NS, LANES = 2, 16, 16
NW = NC * NS
TOK_PW = NTOK // NW
WPB = S_SZ // TOK_PW
T = 16
NCHUNK = TOK_PW // T
ND = LD // LANES


def _proj_body(tabx, taby, tabh, tabw, seq, w_ref, b_ref, out):
    def dot(a, lo):
        return lax.dot_general(
            a[...], w_ref[pl.ds(lo, DPC), :],
            (((1,), (0,)), ((), ())),
            preferred_element_type=jnp.float32,
            precision=lax.Precision.HIGHEST,
        )

    out[pl.ds(0 * ROWS_PER_TAB, ROWS_PER_TAB), :LD] = dot(tabx, 0 * DPC)
    out[pl.ds(1 * ROWS_PER_TAB, ROWS_PER_TAB), :LD] = dot(taby, 1 * DPC)
    out[pl.ds(2 * ROWS_PER_TAB, ROWS_PER_TAB), :LD] = dot(tabx, 2 * DPC)
    out[pl.ds(3 * ROWS_PER_TAB, ROWS_PER_TAB), :LD] = dot(taby, 3 * DPC)
    out[pl.ds(4 * ROWS_PER_TAB, ROWS_PER_TAB), :LD] = dot(tabh, 4 * DPC)
    out[pl.ds(5 * ROWS_PER_TAB, ROWS_PER_TAB), :LD] = (
        dot(tabw, 5 * DPC) + b_ref[...][None, :]
    )
    out[pl.ds(SEQ_BASE, 2048), :LD] = seq[...]
    out[:, LD:] = jnp.zeros((N_TAB, LDP - LD), jnp.float32)


def _proj(tabx, taby, tabh, tabw, seq, W, b):
    return pl.pallas_call(
        _proj_body,
        out_shape=jax.ShapeDtypeStruct((N_TAB, LDP), jnp.float32),
    )(tabx, taby, tabh, tabw, seq, W, b)


def _lane_sum(x):
    idx = lax.iota(jnp.int32, LANES)
    dnums = lax.GatherDimensionNumbers(
        offset_dims=(), collapsed_slice_dims=(0,), start_index_map=(0,))
    for sh in (1, 2, 4, 8):
        perm = lax.gather(
            x, (idx ^ sh)[:, None], dnums, (1,),
            mode=lax.GatherScatterMode.PROMISE_IN_BOUNDS)
        x = x + perm
    return x


_mesh = plsc.VectorSubcoreMesh(core_axis_name="c", subcore_axis_name="s")


@functools.partial(
    pl.kernel,
    out_type=jax.ShapeDtypeStruct((B_SZ, S_SZ, LD), jnp.float32),
    mesh=_mesh,
    scratch_types=[
        pltpu.VMEM((TOK_PW * 4,), jnp.int32),
        pltpu.VMEM((TOK_PW,), jnp.int32),
        pltpu.VMEM((7, TOK_PW), jnp.int32),
        pltpu.VMEM((2, 7, T, LDP), jnp.float32),
        pltpu.VMEM((2, T, LD), jnp.float32),
        pltpu.VMEM((2, LD), jnp.float32),
        pltpu.SemaphoreType.DMA,
        pltpu.SemaphoreType.DMA,
        pltpu.SemaphoreType.DMA,
        pltpu.SemaphoreType.DMA,
    ],
    compiler_params=pltpu.CompilerParams(
        use_tc_tiling_on_sc=True, needs_layout_passes=False),
)
def _sc_body(ptab, bbf, pos, gamma, beta, out,
             braw, posb, idxs, rows, outv, gb, sem_a, sem_b, sem_oa, sem_ob):
    wid = lax.axis_index("s") * NC + lax.axis_index("c")
    wbase = wid * TOK_PW
    brow = wid // WPB
    s_base = (wid % WPB) * TOK_PW

    pltpu.sync_copy(gamma, gb.at[0])
    pltpu.sync_copy(beta, gb.at[1])
    pltpu.sync_copy(bbf.at[pl.ds(wbase * 4, TOK_PW * 4)], braw)
    pltpu.sync_copy(pos.at[pl.ds(wbase, TOK_PW)], posb)

    lane4 = lax.iota(jnp.int32, LANES) * 4
    for i in range(TOK_PW // LANES):
        sl = pl.ds(i * LANES, LANES)
        b0 = plsc.load_gather(braw, [lane4 + (i * 4 * LANES + 0)])
        b1 = plsc.load_gather(braw, [lane4 + (i * 4 * LANES + 1)])
        b2 = plsc.load_gather(braw, [lane4 + (i * 4 * LANES + 2)])
        b3 = plsc.load_gather(braw, [lane4 + (i * 4 * LANES + 3)])
        idxs[0, sl] = b0
        idxs[1, sl] = b1 + ROWS_PER_TAB
        idxs[2, sl] = b2 + 2 * ROWS_PER_TAB
        idxs[3, sl] = b3 + 3 * ROWS_PER_TAB
        idxs[4, sl] = (b3 - b1) + 4 * ROWS_PER_TAB
        idxs[5, sl] = (b2 - b0) + 5 * ROWS_PER_TAB
        idxs[6, sl] = posb[sl] + SEQ_BASE
    row_sems = (sem_a, sem_b)
    out_sems = (sem_oa, sem_ob)

    def fire(c):
        buf = c % 2
        return [pltpu.async_copy(
            ptab.at[idxs.at[j, pl.ds(c * T, T)]],
            rows.at[buf, j], row_sems[buf]) for j in range(7)]

    g_regs = [gb[0, pl.ds(d * LANES, LANES)] for d in range(ND)]
    bt_regs = [gb[1, pl.ds(d * LANES, LANES)] for d in range(ND)]

    pending = {0: fire(0)}
    out_cps = {}
    for c in range(NCHUNK):
        buf = c % 2
        if c + 1 < NCHUNK:
            pending[c + 1] = fire(c + 1)
        for cp in pending.pop(c):
            cp.wait()
        if c >= 2:
            out_cps.pop(c - 2).wait()

        def token_body(t, carry, buf=buf):
            xs = []
            s_acc = None
            q_acc = None
            for d in range(ND):
                sl = pl.ds(d * LANES, LANES)
                x = rows[buf, 0, t, sl]
                for j in range(1, 7):
                    x = x + rows[buf, j, t, sl]
                xs.append(x)
                s_acc = x if d == 0 else s_acc + x
                q_acc = x * x if d == 0 else q_acc + x * x
            inv_n = jnp.float32(1.0 / LD)
            s = _lane_sum(s_acc)
            q = _lane_sum(q_acc)
            mu = s * inv_n
            var = q * inv_n - mu * mu
            x0 = var + jnp.float32(EPS)
            ii = lax.bitcast_convert_type(x0, jnp.int32)
            ii = jnp.int32(0x5F3759DF) - lax.shift_right_logical(ii, 1)
            y = lax.bitcast_convert_type(ii, jnp.float32)
            for _ in range(3):
                y = y * (jnp.float32(1.5) - jnp.float32(0.5) * x0 * y * y)
            for d in range(ND):
                sl = pl.ds(d * LANES, LANES)
                outv[buf, t, sl] = (xs[d] - mu) * y * g_regs[d] + bt_regs[d]
            return carry

        lax.fori_loop(0, T, token_body, 0)
        out_cps[c] = pltpu.async_copy(
            outv.at[buf], out.at[brow, pl.ds(s_base + c * T, T), :],
            out_sems[buf])

    out_cps.pop(NCHUNK - 2).wait()
    out_cps.pop(NCHUNK - 1).wait()


@jax.jit
def kernel(bbox, position_ids, tab_x, tab_y, tab_h, tab_w, seq_tab, W, b,
           gamma, beta):
    n = B_SZ * S_SZ
    bbf = bbox.astype(jnp.int32).reshape(n * 4)
    pos = position_ids.astype(jnp.int32).reshape(n)
    ptab = _proj(tab_x, tab_y, tab_h, tab_w, seq_tab, W, b)
    return _sc_body(ptab, bbf, pos, gamma, beta)
